# Initial kernel scaffold; baseline (speedup 1.0000x reference)
import functools

import jax
import jax.numpy as jnp
from jax import lax
from jax.experimental import pallas as pl
from jax.experimental.pallas import tpu as pltpu
from jax.experimental.pallas import tpu_sc as plsc

N = 1_000_000
C = 10
NBINS = 15
L = 16
NW = 32
TBL = C * L

BASE = 31_248
TAIL = N - BASE * NW
CS = 4_464
NCHUNK = BASE // CS
NG = CS // L


def _sc_body(logits_hbm, labels_hbm, out_hbm, lbuf, labbuf, table, outbuf):
    cid = lax.axis_index("c")
    sid = lax.axis_index("s")
    wid = sid * 2 + cid

    zero = jnp.zeros((L,), jnp.float32)
    for i in range(TBL):
        table[pl.ds(i * L, L)] = zero

    iota = lax.iota(jnp.int32, L)
    lanebase = iota * TBL
    gbase = iota * C

    def do_groups(ngroups):
        def body(g, carry):
            row0 = g * (L * C)
            idx = gbase + row0
            ls = [plsc.load_gather(lbuf, [idx + c]) for c in range(C)]
            m = ls[0]
            for c in range(1, C):
                m = jnp.maximum(m, ls[c])
            es = [jnp.exp(v - m) for v in ls]
            s = es[0]
            for c in range(1, C):
                s = s + es[c]
            r = 1.0 / s
            ylab = labbuf[pl.ds(g * L, L)]
            for c in range(C):
                p = es[c] * r
                z = jnp.where(ylab == c, p - 1.0, p)
                z = jnp.where(p > 0.0, z, 0.0)
                b = jnp.minimum((p * 15.0).astype(jnp.int32), NBINS - 1)
                addr = lanebase + (b + c * L)
                plsc.addupdate_scatter(table, [addr], z)
            return carry

        lax.fori_loop(0, ngroups, body, 0)

    for ch in range(NCHUNK):
        samp0 = wid * BASE + ch * CS
        pltpu.sync_copy(logits_hbm.at[pl.ds(samp0 * C, CS * C)],
                        lbuf.at[pl.ds(0, CS * C)])
        pltpu.sync_copy(labels_hbm.at[pl.ds(samp0, CS)],
                        labbuf.at[pl.ds(0, CS)])
        do_groups(NG)

    @pl.when(wid == NW - 1)
    def _():
        samp0 = BASE * NW
        pltpu.sync_copy(logits_hbm.at[pl.ds(samp0 * C, TAIL * C)],
                        lbuf.at[pl.ds(0, TAIL * C)])
        pltpu.sync_copy(labels_hbm.at[pl.ds(samp0, TAIL)],
                        labbuf.at[pl.ds(0, TAIL)])
        do_groups(TAIL // L)

    for grp in range(C):
        acc = table[pl.ds(grp * L, L)]
        for lane in range(1, L):
            acc = acc + table[pl.ds(lane * TBL + grp * L, L)]
        outbuf[pl.ds(grp * L, L)] = acc
    pltpu.sync_copy(outbuf, out_hbm.at[wid])


def _combine_body(x_ref, o_ref):
    s = jnp.sum(x_ref[...], axis=0)
    o_ref[0, 0] = jnp.sum(jnp.abs(s)) * (1.0 / (N * C))


@jax.jit
def kernel(logits, labels):
    mesh = plsc.VectorSubcoreMesh(core_axis_name="c", subcore_axis_name="s")
    sc = pl.kernel(
        _sc_body,
        mesh=mesh,
        out_type=jax.ShapeDtypeStruct((NW, TBL), jnp.float32),
        scratch_types=[
            pltpu.VMEM((CS * C,), jnp.float32),
            pltpu.VMEM((CS,), jnp.int32),
            pltpu.VMEM((L * TBL,), jnp.float32),
            pltpu.VMEM((TBL,), jnp.float32),
        ],
    )
    part = sc(logits.reshape(-1), labels)
    out = pl.pallas_call(
        _combine_body,
        out_shape=jax.ShapeDtypeStruct((1, 1), jnp.float32),
    )(part)
    return out.reshape((1,))


# speedup vs baseline: 1.0659x; 1.0659x over previous
import functools

import jax
import jax.numpy as jnp
from jax import lax
from jax.experimental import pallas as pl
from jax.experimental.pallas import tpu as pltpu
from jax.experimental.pallas import tpu_sc as plsc

N = 1_000_000
C = 10
NBINS = 15
L = 16
NW = 32
TBL = C * L

BASE = 31_248
TAIL = N - BASE * NW
CS = 4_464
NCHUNK = BASE // CS
NG = CS // L


def _sc_body(logits_hbm, labels_hbm, out_hbm, lbuf, labbuf, table, outbuf):
    cid = lax.axis_index("c")
    sid = lax.axis_index("s")
    wid = sid * 2 + cid

    zero = jnp.zeros((L,), jnp.float32)
    for i in range(TBL):
        table[pl.ds(i * L, L)] = zero

    iota = lax.iota(jnp.int32, L)
    lanebase = iota * TBL
    gbase = iota * C

    def do_groups(ngroups):
        def body(g, carry):
            row0 = g * (L * C)
            idx = gbase + row0
            ls = [plsc.load_gather(lbuf, [idx + c]) for c in range(C)]
            m = ls[0]
            for c in range(1, C):
                m = jnp.maximum(m, ls[c])
            es = [jnp.exp(v - m) for v in ls]
            s = es[0]
            for c in range(1, C):
                s = s + es[c]
            r = 1.0 / s
            ylab = labbuf[pl.ds(g * L, L)]
            for c in range(C):
                p = es[c] * r
                z = jnp.where(ylab == c, p - 1.0, p)
                z = jnp.where(p > 0.0, z, 0.0)
                b = jnp.minimum((p * 15.0).astype(jnp.int32), NBINS - 1)
                addr = lanebase + (b + c * L)
                plsc.addupdate_scatter(table, [addr], z)
            return carry

        lax.fori_loop(0, ngroups, body, 0)

    for ch in range(NCHUNK):
        samp0 = wid * BASE + ch * CS
        pltpu.sync_copy(logits_hbm.at[pl.ds(samp0 * C, CS * C)],
                        lbuf.at[pl.ds(0, CS * C)])
        pltpu.sync_copy(labels_hbm.at[pl.ds(samp0, CS)],
                        labbuf.at[pl.ds(0, CS)])
        do_groups(NG)

    @pl.when(wid == NW - 1)
    def _():
        samp0 = BASE * NW
        pltpu.sync_copy(logits_hbm.at[pl.ds(samp0 * C, TAIL * C)],
                        lbuf.at[pl.ds(0, TAIL * C)])
        pltpu.sync_copy(labels_hbm.at[pl.ds(samp0, TAIL)],
                        labbuf.at[pl.ds(0, TAIL)])
        do_groups(TAIL // L)

    for grp in range(C):
        acc = table[pl.ds(grp * L, L)]
        for lane in range(1, L):
            acc = acc + table[pl.ds(lane * TBL + grp * L, L)]
        outbuf[pl.ds(grp * L, L)] = acc
    pltpu.sync_copy(outbuf, out_hbm.at[wid])


def _combine_body(x_ref, o_ref):
    s = jnp.sum(x_ref[...], axis=0)
    tot = jnp.sum(jnp.abs(s)) * (1.0 / (N * C))
    o_ref[...] = tot.reshape(1, 1)


@jax.jit
def kernel(logits, labels):
    mesh = plsc.VectorSubcoreMesh(core_axis_name="c", subcore_axis_name="s")
    sc = pl.kernel(
        _sc_body,
        mesh=mesh,
        compiler_params=pltpu.CompilerParams(needs_layout_passes=False),
        out_type=jax.ShapeDtypeStruct((NW, TBL), jnp.float32),
        scratch_types=[
            pltpu.VMEM((CS * C,), jnp.float32),
            pltpu.VMEM((CS,), jnp.int32),
            pltpu.VMEM((L * TBL,), jnp.float32),
            pltpu.VMEM((TBL,), jnp.float32),
        ],
    )
    part = sc(logits.reshape(-1), labels)
    out = pl.pallas_call(
        _combine_body,
        out_shape=jax.ShapeDtypeStruct((1, 1), jnp.float32),
    )(part)
    return out.reshape((1,))
